# Initial kernel scaffold; baseline (speedup 1.0000x reference)
import jax
import jax.numpy as jnp
from jax.experimental import pallas as pl


def kernel(selection_probabilities, parameter, selection_index):
    raise NotImplementedError("write your pallas kernel here")


# speedup vs baseline: 1.1818x; 1.1818x over previous
import jax
import jax.numpy as jnp
from jax.experimental import pallas as pl

_BANK = 64
_ROWS_BLOCK = 8192


def _combine_kernel(probs_ref, idx_ref, param_ref, out_ref):
    probs = probs_ref[...]
    idx = idx_ref[...]
    b, k = probs.shape
    lanes = jax.lax.broadcasted_iota(jnp.int32, (b, _BANK), 1)
    w = jnp.zeros((b, _BANK), jnp.float32)
    for kk in range(k):
        w = w + jnp.where(idx[:, kk:kk + 1] == lanes, probs[:, kk:kk + 1], 0.0)
    out_ref[...] = jax.lax.dot_general(
        w, param_ref[...], (((1,), (1,)), ((), ())),
        preferred_element_type=jnp.float32)


def kernel(selection_probabilities, parameter, selection_index):
    s0, s1, bank = parameter.shape
    b, k = selection_index.shape
    rows = s0 * s1
    flat = parameter.reshape(rows, bank)
    grid = rows // _ROWS_BLOCK
    out = pl.pallas_call(
        _combine_kernel,
        grid=(grid,),
        in_specs=[
            pl.BlockSpec((b, k), lambda i: (0, 0)),
            pl.BlockSpec((b, k), lambda i: (0, 0)),
            pl.BlockSpec((_ROWS_BLOCK, bank), lambda i: (i, 0)),
        ],
        out_specs=pl.BlockSpec((b, _ROWS_BLOCK), lambda i: (0, i)),
        out_shape=jax.ShapeDtypeStruct((b, rows), jnp.float32),
    )(selection_probabilities, selection_index, flat)
    return out.reshape(b, s0, s1)


# speedup vs baseline: 1.2623x; 1.0681x over previous
import jax
import jax.numpy as jnp
from jax.experimental import pallas as pl

_BANK = 64
_ROWS_BLOCK = 32768


def _combine_kernel(probs_ref, idx_ref, param_ref, out_ref):
    probs = probs_ref[...]
    idx = idx_ref[...]
    b, k = probs.shape
    lanes = jax.lax.broadcasted_iota(jnp.int32, (b, _BANK), 1)
    w = jnp.zeros((b, _BANK), jnp.float32)
    for kk in range(k):
        w = w + jnp.where(idx[:, kk:kk + 1] == lanes, probs[:, kk:kk + 1], 0.0)
    out_ref[...] = jax.lax.dot_general(
        w, param_ref[...], (((1,), (1,)), ((), ())),
        preferred_element_type=jnp.float32)


def kernel(selection_probabilities, parameter, selection_index):
    s0, s1, bank = parameter.shape
    b, k = selection_index.shape
    rows = s0 * s1
    flat = parameter.reshape(rows, bank)
    grid = rows // _ROWS_BLOCK
    out = pl.pallas_call(
        _combine_kernel,
        grid=(grid,),
        in_specs=[
            pl.BlockSpec((b, k), lambda i: (0, 0)),
            pl.BlockSpec((b, k), lambda i: (0, 0)),
            pl.BlockSpec((_ROWS_BLOCK, bank), lambda i: (i, 0)),
        ],
        out_specs=pl.BlockSpec((b, _ROWS_BLOCK), lambda i: (0, i)),
        out_shape=jax.ShapeDtypeStruct((b, rows), jnp.float32),
    )(selection_probabilities, selection_index, flat)
    return out.reshape(b, s0, s1)


# speedup vs baseline: 3.5824x; 2.8380x over previous
import jax
import jax.numpy as jnp
from jax.experimental import pallas as pl

_BANK = 64
_BS0 = 8


def _combine_kernel(probs_ref, idx_ref, param_ref, out_ref):
    probs = probs_ref[...]
    idx = idx_ref[...]
    b, k = probs.shape
    lanes = jax.lax.broadcasted_iota(jnp.int32, (b, _BANK), 1)
    w = jnp.zeros((b, _BANK), jnp.float32)
    for kk in range(k):
        w = w + jnp.where(idx[:, kk:kk + 1] == lanes, probs[:, kk:kk + 1], 0.0)
    v = param_ref[...]
    for i in range(v.shape[0]):
        out_ref[:, i, :] = jax.lax.dot_general(
            w, v[i], (((1,), (0,)), ((), ())),
            preferred_element_type=jnp.float32)


def kernel(selection_probabilities, parameter, selection_index):
    s0, s1, bank = parameter.shape
    b, k = selection_index.shape
    pview = jnp.transpose(parameter, (0, 2, 1))
    grid = s0 // _BS0
    out = pl.pallas_call(
        _combine_kernel,
        grid=(grid,),
        in_specs=[
            pl.BlockSpec((b, k), lambda i: (0, 0)),
            pl.BlockSpec((b, k), lambda i: (0, 0)),
            pl.BlockSpec((_BS0, bank, s1), lambda i: (i, 0, 0)),
        ],
        out_specs=pl.BlockSpec((b, _BS0, s1), lambda i: (0, i, 0)),
        out_shape=jax.ShapeDtypeStruct((b, s0, s1), jnp.float32),
    )(selection_probabilities, selection_index, pview)
    return out


# speedup vs baseline: 5.5044x; 1.5365x over previous
import jax
import jax.numpy as jnp
from jax.experimental import pallas as pl

_BANK = 64
_BS0 = 32


def _combine_kernel(probs_ref, idx_ref, param_ref, out_ref):
    probs = probs_ref[...]
    idx = idx_ref[...]
    b, k = probs.shape
    lanes = jax.lax.broadcasted_iota(jnp.int32, (b, _BANK), 1)
    w = jnp.zeros((b, _BANK), jnp.float32)
    for kk in range(k):
        w = w + jnp.where(idx[:, kk:kk + 1] == lanes, probs[:, kk:kk + 1], 0.0)
    v = param_ref[...]
    for i in range(v.shape[0]):
        out_ref[:, i, :] = jax.lax.dot_general(
            w, v[i], (((1,), (0,)), ((), ())),
            preferred_element_type=jnp.float32)


def kernel(selection_probabilities, parameter, selection_index):
    s0, s1, bank = parameter.shape
    b, k = selection_index.shape
    pview = jnp.transpose(parameter, (0, 2, 1))
    grid = s0 // _BS0
    out = pl.pallas_call(
        _combine_kernel,
        grid=(grid,),
        in_specs=[
            pl.BlockSpec((b, k), lambda i: (0, 0)),
            pl.BlockSpec((b, k), lambda i: (0, 0)),
            pl.BlockSpec((_BS0, bank, s1), lambda i: (i, 0, 0)),
        ],
        out_specs=pl.BlockSpec((b, _BS0, s1), lambda i: (0, i, 0)),
        out_shape=jax.ShapeDtypeStruct((b, s0, s1), jnp.float32),
    )(selection_probabilities, selection_index, pview)
    return out
